# Initial kernel scaffold; baseline (speedup 1.0000x reference)
import jax
import jax.numpy as jnp
from jax.experimental import pallas as pl


def kernel(x, router_w, w1, b1, w2, b2):
    raise NotImplementedError("write your pallas kernel here")


# speedup vs baseline: 1.0672x; 1.0672x over previous
import functools

import jax
import jax.numpy as jnp
from jax.experimental import pallas as pl
from jax.experimental.pallas import tpu as pltpu


_SUB = 1024


def _top2_weights(probs):
    lane = jax.lax.broadcasted_iota(jnp.int32, probs.shape, 1)
    nlanes = jnp.int32(probs.shape[1])
    m1 = jnp.max(probs, axis=-1, keepdims=True)
    a1 = jnp.min(jnp.where(probs == m1, lane, nlanes), axis=-1, keepdims=True)
    probs2 = jnp.where(lane == a1, -1.0, probs)
    m2 = jnp.max(probs2, axis=-1, keepdims=True)
    a2 = jnp.min(jnp.where(probs2 == m2, lane, nlanes), axis=-1, keepdims=True)
    denom = m1 + m2 + 1e-9
    return jnp.where(lane == a1, m1 / denom,
                     jnp.where(lane == a2, m2 / denom, 0.0))


def _moe_dense_body(x_ref, rw_ref, w1_ref, b1_ref, w2_ref, b2_ref,
                    out_ref, wfull_ref):
    e = pl.program_id(0)
    f = pl.program_id(1)
    T = x_ref.shape[0]
    sub = min(_SUB, T)
    ns = T // sub

    @pl.when((e == 0) & (f == 0))
    def _router():
        def step(i, _):
            sl = pl.ds(i * sub, sub)
            logits = jax.lax.dot_general(
                x_ref[sl, :], rw_ref[...], (((1,), (1,)), ((), ())),
                preferred_element_type=jnp.float32)
            wfull_ref[sl, :] = _top2_weights(jax.nn.softmax(logits, axis=-1))
            out_ref[sl, :] = jnp.zeros((sub, out_ref.shape[1]), jnp.float32)
            return 0
        jax.lax.fori_loop(0, ns, step, 0)

    def step(i, _):
        sl = pl.ds(i * sub, sub)
        wf = wfull_ref[sl, :]
        lane = jax.lax.broadcasted_iota(jnp.int32, wf.shape, 1)
        wt = jnp.sum(jnp.where(lane == e, wf, 0.0), axis=-1, keepdims=True)
        h = jax.lax.dot_general(
            x_ref[sl, :], w1_ref[0], (((1,), (1,)), ((), ())),
            preferred_element_type=jnp.float32)
        h = jnp.maximum(h + b1_ref[0], 0.0) * wt
        acc = jax.lax.dot_general(
            h, w2_ref[0], (((1,), (1,)), ((), ())),
            preferred_element_type=jnp.float32)
        acc = jnp.where(f == 0, acc + wt * b2_ref[0], acc)
        out_ref[sl, :] += acc
        return 0
    jax.lax.fori_loop(0, ns, step, 0)


def _moe_dense(x_flat, router_w, w1, b1, w2, b2, *, interpret=False):
    T, D = x_flat.shape
    E, F, _ = w1.shape
    FCH = min(512, F)
    NF = F // FCH
    b1r = b1.reshape(E * NF, 1, FCH)
    b2r = b2.reshape(E, 1, D)

    return pl.pallas_call(
        _moe_dense_body,
        grid=(E, NF),
        in_specs=[
            pl.BlockSpec((T, D), lambda e, f: (0, 0)),
            pl.BlockSpec((E, D), lambda e, f: (0, 0)),
            pl.BlockSpec((1, FCH, D), lambda e, f: (e, f, 0)),
            pl.BlockSpec((1, 1, FCH), lambda e, f, _nf=NF: (e * _nf + f, 0, 0)),
            pl.BlockSpec((1, D, FCH), lambda e, f: (e, 0, f)),
            pl.BlockSpec((1, 1, D), lambda e, f: (e, 0, 0)),
        ],
        out_specs=pl.BlockSpec((T, D), lambda e, f: (0, 0)),
        out_shape=jax.ShapeDtypeStruct((T, D), jnp.float32),
        scratch_shapes=[
            pltpu.VMEM((T, E), jnp.float32),
        ],
        interpret=interpret,
    )(x_flat, router_w, w1, b1r, w2, b2r)


def kernel(x, router_w, w1, b1, w2, b2):
    B, N, D = x.shape
    x_flat = x.reshape(B * N, D)
    out = _moe_dense(x_flat, router_w, w1, b1, w2, b2)
    return out.reshape(B, N, D)


# speedup vs baseline: 1.0818x; 1.0137x over previous
import functools

import jax
import jax.numpy as jnp
from jax.experimental import pallas as pl
from jax.experimental.pallas import tpu as pltpu


_SUB = 1024


def _top2_weights(probs):
    lane = jax.lax.broadcasted_iota(jnp.int32, probs.shape, 1)
    nlanes = jnp.int32(probs.shape[1])
    m1 = jnp.max(probs, axis=-1, keepdims=True)
    a1 = jnp.min(jnp.where(probs == m1, lane, nlanes), axis=-1, keepdims=True)
    probs2 = jnp.where(lane == a1, -1.0, probs)
    m2 = jnp.max(probs2, axis=-1, keepdims=True)
    a2 = jnp.min(jnp.where(probs2 == m2, lane, nlanes), axis=-1, keepdims=True)
    denom = m1 + m2 + 1e-9
    return jnp.where(lane == a1, m1 / denom,
                     jnp.where(lane == a2, m2 / denom, 0.0))


def _moe_dense_body(x_ref, rw_ref, w1_ref, b1_ref, w2_ref, b2_ref,
                    out_ref, wfull_ref):
    e = pl.program_id(0)
    f = pl.program_id(1)
    T = x_ref.shape[0]
    sub = min(_SUB, T)
    ns = T // sub

    @pl.when((e == 0) & (f == 0))
    def _router():
        def step(i, _):
            sl = pl.ds(i * sub, sub)
            logits = jax.lax.dot_general(
                x_ref[sl, :], rw_ref[...], (((1,), (1,)), ((), ())),
                preferred_element_type=jnp.float32)
            wfull_ref[sl, :] = _top2_weights(jax.nn.softmax(logits, axis=-1))
            out_ref[sl, :] = jnp.zeros((sub, out_ref.shape[1]), jnp.float32)
            return 0
        jax.lax.fori_loop(0, ns, step, 0)

    def step(i, _):
        sl = pl.ds(i * sub, sub)
        wf = wfull_ref[sl, :]
        lane = jax.lax.broadcasted_iota(jnp.int32, wf.shape, 1)
        wt = jnp.sum(jnp.where(lane == e, wf, 0.0), axis=-1, keepdims=True)
        h = jax.lax.dot_general(
            x_ref[sl, :], w1_ref[0], (((1,), (1,)), ((), ())),
            preferred_element_type=jnp.float32)
        h = jnp.maximum(h + b1_ref[0], 0.0) * wt
        acc = jax.lax.dot_general(
            h, w2_ref[0], (((1,), (1,)), ((), ())),
            preferred_element_type=jnp.float32)
        acc = jnp.where(f == 0, acc + wt * b2_ref[0], acc)
        out_ref[sl, :] += acc
        return 0
    jax.lax.fori_loop(0, ns, step, 0)


def _moe_dense(x_flat, router_w, w1, b1, w2, b2, *, interpret=False):
    T, D = x_flat.shape
    E, F, _ = w1.shape
    FCH = min(512, F)
    NF = F // FCH
    b1r = b1.reshape(E * NF, 1, FCH)
    b2r = b2.reshape(E, 1, D)

    return pl.pallas_call(
        _moe_dense_body,
        grid=(E, NF),
        in_specs=[
            pl.BlockSpec((T, D), lambda e, f: (0, 0)),
            pl.BlockSpec((E, D), lambda e, f: (0, 0)),
            pl.BlockSpec((1, FCH, D), lambda e, f: (e, f, 0)),
            pl.BlockSpec((1, 1, FCH), lambda e, f, _nf=NF: (e * _nf + f, 0, 0)),
            pl.BlockSpec((1, D, FCH), lambda e, f: (e, 0, f)),
            pl.BlockSpec((1, 1, D), lambda e, f: (e, 0, 0)),
        ],
        out_specs=pl.BlockSpec((T, D), lambda e, f: (0, 0)),
        out_shape=jax.ShapeDtypeStruct((T, D), jnp.float32),
        scratch_shapes=[
            pltpu.VMEM((T, E), jnp.float32),
        ],
        interpret=interpret,
    )(x_flat, router_w, w1, b1r, w2, b2r)


def kernel(x, router_w, w1, b1, w2, b2):
    import pipe_dev as _pd2
    return _pd2.moe_routed(x, router_w, w1, b1, w2, b2)


import pipe_dev as _pd


def kernel_routed_dev(x, router_w, w1, b1, w2, b2):
    return _pd.moe_routed(x, router_w, w1, b1, w2, b2)
